# Initial kernel scaffold; baseline (speedup 1.0000x reference)
import functools

import jax
import jax.numpy as jnp
from jax import lax
from jax.experimental import pallas as pl
from jax.experimental.pallas import tpu as pltpu
from jax.experimental.pallas import tpu_sc as plsc


@functools.lru_cache(maxsize=None)
def _make_lookup(B, V, D):
    info = plsc.get_sparse_core_info()
    NC, NS = info.num_cores, info.num_subcores
    NW = NC * NS
    assert B % NW == 0
    b_per_w = B // NW

    GATHER = 128
    SUB = 10
    CHUNK = GATHER * SUB
    assert b_per_w % CHUNK == 0
    n_chunks = b_per_w // CHUNK

    mesh = plsc.VectorSubcoreMesh(core_axis_name="c", subcore_axis_name="s")

    @functools.partial(
        pl.kernel,
        mesh=mesh,
        out_type=jax.ShapeDtypeStruct((B, D), jnp.float32),
        scratch_types=[
            pltpu.VMEM((b_per_w,), jnp.int32),
            pltpu.VMEM((CHUNK, D), jnp.float32),
            pltpu.SemaphoreType.DMA,
        ],
    )
    def lookup(idx_hbm, table_hbm, out_hbm, idx_v, rows, gsem):
        wid = lax.axis_index("s") * NC + lax.axis_index("c")
        base = wid * b_per_w
        pltpu.sync_copy(idx_hbm.at[pl.ds(base, b_per_w)], idx_v)

        def body(c, _):
            off = c * CHUNK
            cps = []
            for j in range(SUB):
                cps.append(
                    pltpu.async_copy(
                        table_hbm.at[idx_v.at[pl.ds(off + j * GATHER, GATHER)]],
                        rows.at[pl.ds(j * GATHER, GATHER)],
                        gsem,
                    )
                )
            for cp in cps:
                cp.wait()
            pltpu.sync_copy(rows, out_hbm.at[pl.ds(base + off, CHUNK)])
            return 0

        lax.fori_loop(0, n_chunks, body, 0)

    return lookup


def kernel(token_ids, weight):
    S0, S1 = token_ids.shape
    V, D = weight.shape
    B = S0 * S1
    flat = token_ids.reshape(B).astype(jnp.int32)
    out = _make_lookup(B, V, D)(flat, weight)
    return out.reshape(S0, S1, D)


# speedup vs baseline: 1.1042x; 1.1042x over previous
import functools

import jax
import jax.numpy as jnp
from jax import lax
from jax.experimental import pallas as pl
from jax.experimental.pallas import tpu as pltpu
from jax.experimental.pallas import tpu_sc as plsc


@functools.lru_cache(maxsize=None)
def _make_lookup(B, V, D):
    info = plsc.get_sparse_core_info()
    NC, NS = info.num_cores, info.num_subcores
    NW = NC * NS
    assert B % NW == 0
    b_per_w = B // NW

    GATHER = 128
    SUB = 10
    CHUNK = GATHER * SUB
    assert b_per_w % CHUNK == 0
    n_chunks = b_per_w // CHUNK

    mesh = plsc.VectorSubcoreMesh(core_axis_name="c", subcore_axis_name="s")

    @functools.partial(
        pl.kernel,
        mesh=mesh,
        compiler_params=pltpu.CompilerParams(use_tc_tiling_on_sc=False),
        out_type=jax.ShapeDtypeStruct((B, D), jnp.float32),
        scratch_types=[
            pltpu.VMEM((b_per_w,), jnp.int32),
            pltpu.VMEM((CHUNK, D), jnp.float32),
            pltpu.SemaphoreType.DMA,
        ],
    )
    def lookup(idx_hbm, table_hbm, out_hbm, idx_v, rows, gsem):
        wid = lax.axis_index("s") * NC + lax.axis_index("c")
        base = wid * b_per_w
        pltpu.sync_copy(idx_hbm.at[pl.ds(base, b_per_w)], idx_v)

        def body(c, _):
            off = c * CHUNK
            cps = []
            for j in range(SUB):
                cps.append(
                    pltpu.async_copy(
                        table_hbm.at[idx_v.at[pl.ds(off + j * GATHER, GATHER)]],
                        rows.at[pl.ds(j * GATHER, GATHER)],
                        gsem,
                    )
                )
            for cp in cps:
                cp.wait()
            pltpu.sync_copy(rows, out_hbm.at[pl.ds(base + off, CHUNK)])
            return 0

        lax.fori_loop(0, n_chunks, body, 0)

    return lookup


def kernel(token_ids, weight):
    S0, S1 = token_ids.shape
    V, D = weight.shape
    B = S0 * S1
    flat = token_ids.reshape(B).astype(jnp.int32)
    out = _make_lookup(B, V, D)(flat, weight)
    return out.reshape(S0, S1, D)


# speedup vs baseline: 1.1105x; 1.0058x over previous
import functools

import jax
import jax.numpy as jnp
from jax import lax
from jax.experimental import pallas as pl
from jax.experimental.pallas import tpu as pltpu
from jax.experimental.pallas import tpu_sc as plsc


@functools.lru_cache(maxsize=None)
def _make_lookup(B, V, D):
    info = plsc.get_sparse_core_info()
    NC, NS = info.num_cores, info.num_subcores
    NW = NC * NS
    assert B % NW == 0
    b_per_w = B // NW

    GATHER = 128
    SUB = 10
    CHUNK = GATHER * SUB
    assert b_per_w % CHUNK == 0
    n_chunks = b_per_w // CHUNK

    mesh = plsc.VectorSubcoreMesh(core_axis_name="c", subcore_axis_name="s")

    @functools.partial(
        pl.kernel,
        mesh=mesh,
        compiler_params=pltpu.CompilerParams(use_tc_tiling_on_sc=False),
        out_type=jax.ShapeDtypeStruct((B, D), jnp.float32),
        scratch_types=[
            pltpu.VMEM((b_per_w,), jnp.int32),
            pltpu.VMEM((CHUNK, D), jnp.float32),
            pltpu.VMEM((CHUNK, D), jnp.float32),
            pltpu.SemaphoreType.DMA,
            pltpu.SemaphoreType.DMA,
        ],
    )
    def lookup(idx_hbm, table_hbm, out_hbm, idx_v, rows0, rows1, gsem, wsem):
        wid = lax.axis_index("s") * NC + lax.axis_index("c")
        base = wid * b_per_w
        pltpu.sync_copy(idx_hbm.at[pl.ds(base, b_per_w)], idx_v)
        bufs = (rows0, rows1)

        def gather_chunk(c, buf):
            cps = []
            for j in range(SUB):
                cps.append(
                    pltpu.async_copy(
                        table_hbm.at[idx_v.at[pl.ds(c * CHUNK + j * GATHER, GATHER)]],
                        buf.at[pl.ds(j * GATHER, GATHER)],
                        gsem,
                    )
                )
            return cps

        def write_chunk(c, buf):
            return pltpu.async_copy(
                buf, out_hbm.at[pl.ds(base + c * CHUNK, CHUNK)], wsem
            )

        def drain_write(buf):
            pltpu.make_async_copy(buf, out_hbm.at[pl.ds(base, CHUNK)], wsem).wait()

        for cp in gather_chunk(0, bufs[0]):
            cp.wait()
        write_chunk(0, bufs[0])
        for cp in gather_chunk(1, bufs[1]):
            cp.wait()
        write_chunk(1, bufs[1])

        def pair_body(p, _):
            for b in range(2):
                c = 2 * p + b
                buf = bufs[b]
                drain_write(buf)
                for cp in gather_chunk(c, buf):
                    cp.wait()
                write_chunk(c, buf)
            return 0

        lax.fori_loop(1, n_chunks // 2, pair_body, 0)
        drain_write(bufs[0])
        drain_write(bufs[1])

    return lookup


def kernel(token_ids, weight):
    S0, S1 = token_ids.shape
    V, D = weight.shape
    B = S0 * S1
    flat = token_ids.reshape(B).astype(jnp.int32)
    out = _make_lookup(B, V, D)(flat, weight)
    return out.reshape(S0, S1, D)


# speedup vs baseline: 1.5424x; 1.3888x over previous
import functools

import jax
import jax.numpy as jnp
from jax import lax
from jax.experimental import pallas as pl
from jax.experimental.pallas import tpu as pltpu
from jax.experimental.pallas import tpu_sc as plsc


@functools.lru_cache(maxsize=None)
def _make_lookup(S0, S1, V, D):
    info = plsc.get_sparse_core_info()
    NC, NS, VL = info.num_cores, info.num_subcores, info.num_lanes
    NW = NC * NS
    LT = 128
    DR = D // 8
    n_lt = S0 // LT
    lt_per_w = n_lt // NW
    n_units = S1 * lt_per_w
    b_per_w = n_units * LT
    assert S0 % (LT * NW) == 0 and D % 8 == 0 and n_units % 2 == 0

    mesh = plsc.VectorSubcoreMesh(core_axis_name="c", subcore_axis_name="s")

    @functools.partial(
        pl.kernel,
        mesh=mesh,
        compiler_params=pltpu.CompilerParams(
            use_tc_tiling_on_sc=False, needs_layout_passes=False
        ),
        out_type=jax.ShapeDtypeStruct((S1, DR, n_lt, 8, LT), jnp.float32),
        scratch_types=[
            pltpu.VMEM((b_per_w,), jnp.int32),
            pltpu.VMEM((LT, D), jnp.float32),
            pltpu.VMEM((LT, D), jnp.float32),
            pltpu.VMEM((DR, 8, LT), jnp.float32),
            pltpu.VMEM((DR, 8, LT), jnp.float32),
            pltpu.SemaphoreType.DMA,
            pltpu.SemaphoreType.DMA,
        ],
    )
    def lookup(idx_hbm, table_hbm, out_hbm, idx_v, r0, r1, t0, t1, gsem, wsem):
        wid = lax.axis_index("s") * NC + lax.axis_index("c")
        s0_base = wid * (lt_per_w * LT)
        rbuf = (r0, r1)
        tbuf = (t0, t1)

        stage = []
        for s1 in range(S1):
            stage.append(
                pltpu.async_copy(
                    idx_hbm.at[pl.ds(s1 * S0 + s0_base, lt_per_w * LT)],
                    idx_v.at[pl.ds(s1 * (lt_per_w * LT), lt_per_w * LT)],
                    gsem,
                )
            )
        for cp in stage:
            cp.wait()

        def gather_unit(u, buf):
            return pltpu.async_copy(table_hbm.at[idx_v.at[pl.ds(u * LT, LT)]], buf, gsem)

        def write_unit(u, buf):
            s1 = u >> 2
            cl = u & 3
            return pltpu.async_copy(
                buf, out_hbm.at[s1, :, wid * lt_per_w + cl], wsem
            )

        def drain_write():
            pltpu.make_async_copy(t0, out_hbm.at[0, :, 0], wsem).wait()

        iota = lax.iota(jnp.int32, VL)

        def transpose_unit(rb, tb):
            for d in range(D):
                col = jnp.full((VL,), d, jnp.int32)
                for g in range(LT // VL):
                    v = plsc.load_gather(rb, [iota + (g * VL), col])
                    tb[d // 8, d % 8, pl.ds(g * VL, VL)] = v

        gps = [gather_unit(0, rbuf[0]), gather_unit(1, rbuf[1])]

        def pair_body(p, _):
            for b in range(2):
                u = 2 * p + b
                gps[b].wait()

                @pl.when(u >= 2)
                def _():
                    drain_write()

                transpose_unit(rbuf[b], tbuf[b])

                @pl.when(u + 2 < n_units)
                def _():
                    gather_unit(u + 2, rbuf[b])

                write_unit(u, tbuf[b])
            return 0

        lax.fori_loop(0, n_units // 2, pair_body, 0, unroll=False)
        drain_write()
        drain_write()

    return lookup


def kernel(token_ids, weight):
    S0, S1 = token_ids.shape
    V, D = weight.shape
    tokens_lin = token_ids.T.reshape(S0 * S1)
    wlin = jax.lax.optimization_barrier(weight.reshape(V * D))
    w2 = wlin.reshape(V, D)
    out5 = _make_lookup(S0, S1, V, D)(tokens_lin, w2)
    return out5.transpose(2, 4, 0, 1, 3).reshape(S0, S1, D)


# speedup vs baseline: 1.9209x; 1.2454x over previous
import functools

import jax
import jax.numpy as jnp
from jax import lax
from jax.experimental import pallas as pl
from jax.experimental.pallas import tpu as pltpu
from jax.experimental.pallas import tpu_sc as plsc


@functools.lru_cache(maxsize=None)
def _make_lookup(S0, S1, V, D):
    info = plsc.get_sparse_core_info()
    NC, NS, VL = info.num_cores, info.num_subcores, info.num_lanes
    NW = NC * NS
    LT = 128
    DR = D // 8
    n_lt = S0 // LT
    lt_per_w = n_lt // NW
    n_units = S1 * lt_per_w
    b_per_w = n_units * LT
    assert S0 % (LT * NW) == 0 and D % 8 == 0 and n_units % 2 == 0

    mesh = plsc.VectorSubcoreMesh(core_axis_name="c", subcore_axis_name="s")

    @functools.partial(
        pl.kernel,
        mesh=mesh,
        compiler_params=pltpu.CompilerParams(
            use_tc_tiling_on_sc=False, needs_layout_passes=False
        ),
        out_type=jax.ShapeDtypeStruct((S1, DR, n_lt, 8, LT), jnp.float32),
        scratch_types=[
            pltpu.VMEM((b_per_w,), jnp.int32),
            pltpu.VMEM((LT, D), jnp.float32),
            pltpu.VMEM((LT, D), jnp.float32),
            pltpu.VMEM((DR, 8, LT), jnp.float32),
            pltpu.VMEM((DR, 8, LT), jnp.float32),
            pltpu.SemaphoreType.DMA,
            pltpu.SemaphoreType.DMA,
        ],
    )
    def lookup(idx_hbm, table_hbm, out_hbm, idx_v, r0, r1, t0, t1, gsem, wsem):
        wid = lax.axis_index("s") * NC + lax.axis_index("c")
        s0_base = wid * (lt_per_w * LT)
        rbuf = (r0, r1)
        tbuf = (t0, t1)

        stage = []
        for s1 in range(S1):
            stage.append(
                pltpu.async_copy(
                    idx_hbm.at[pl.ds(s1 * S0 + s0_base, lt_per_w * LT)],
                    idx_v.at[pl.ds(s1 * (lt_per_w * LT), lt_per_w * LT)],
                    gsem,
                )
            )
        for cp in stage:
            cp.wait()

        def gather_unit(u, buf):
            return pltpu.async_copy(table_hbm.at[idx_v.at[pl.ds(u * LT, LT)]], buf, gsem)

        def write_unit(u, buf):
            s1 = u >> 2
            cl = u & 3
            return pltpu.async_copy(
                buf, out_hbm.at[s1, :, wid * lt_per_w + cl], wsem
            )

        def drain_write():
            pltpu.make_async_copy(t0, out_hbm.at[0, :, 0], wsem).wait()

        iota = lax.iota(jnp.int32, VL)
        n_tr = D * (LT // VL)

        def transpose_unit(rb, tb):
            @plsc.parallel_loop(0, n_tr, 1, unroll=8)
            def _(i):
                d = i >> 3
                g = i & 7
                col = jnp.full((VL,), 0, jnp.int32) + d
                v = plsc.load_gather(rb, [iota + (g << 4), col])
                tb[d >> 3, d & 7, pl.ds(g << 4, VL)] = v

        gps = [gather_unit(0, rbuf[0]), gather_unit(1, rbuf[1])]

        def pair_body(p, _):
            for b in range(2):
                u = 2 * p + b
                gps[b].wait()

                @pl.when(u >= 2)
                def _():
                    drain_write()

                transpose_unit(rbuf[b], tbuf[b])

                @pl.when(u + 2 < n_units)
                def _():
                    gather_unit(u + 2, rbuf[b])

                write_unit(u, tbuf[b])
            return 0

        lax.fori_loop(0, n_units // 2, pair_body, 0, unroll=False)
        drain_write()
        drain_write()

    return lookup


def kernel(token_ids, weight):
    S0, S1 = token_ids.shape
    V, D = weight.shape
    tokens_lin = token_ids.T.reshape(S0 * S1)
    wlin = jax.lax.optimization_barrier(weight.reshape(V * D))
    w2 = wlin.reshape(V, D)
    out5 = _make_lookup(S0, S1, V, D)(tokens_lin, w2)
    return out5.transpose(2, 4, 0, 1, 3).reshape(S0, S1, D)
